# Initial kernel scaffold; baseline (speedup 1.0000x reference)
import jax
import jax.numpy as jnp
from jax.experimental import pallas as pl


def kernel(diffusion_step, embedding, W1, b1, W2, b2):
    raise NotImplementedError("write your pallas kernel here")


# speedup vs baseline: 1.6446x; 1.6446x over previous
import jax
import jax.numpy as jnp
from jax import lax
from jax.experimental import pallas as pl
from jax.experimental.pallas import tpu as pltpu
from jax.experimental.pallas import tpu_sc as plsc

IN_DIM = 256
HIDDEN = 1024
TABLE_PAD = 1024
BATCH = 16384

NC, NS = 2, 16
NW = NC * NS
B_PER_W = BATCH // NW
CHUNK = 64
N_CHUNKS = B_PER_W // CHUNK


def _mlp_body(emb_ref, w1_ref, b1_ref, w2_ref, b2_ref, out_ref):
    h = jnp.dot(emb_ref[...], w1_ref[...], preferred_element_type=jnp.float32)
    h = h + b1_ref[...]
    h = h * jax.nn.sigmoid(h)
    o = jnp.dot(h, w2_ref[...], preferred_element_type=jnp.float32)
    o = o + b2_ref[...]
    out_ref[...] = o * jax.nn.sigmoid(o)


def _gather_body(table_hbm, idx_hbm, out_hbm, idx_v, rows_v, sem):
    wid = lax.axis_index("s") * NC + lax.axis_index("c")
    base = wid * B_PER_W
    pltpu.sync_copy(idx_hbm.at[wid], idx_v)
    for j in range(N_CHUNKS):
        pltpu.async_copy(table_hbm.at[idx_v.at[j]], rows_v, sem).wait()
        pltpu.sync_copy(rows_v, out_hbm.at[pl.ds(base + j * CHUNK, CHUNK)])


def kernel(diffusion_step, embedding, W1, b1, W2, b2):
    emb = jnp.pad(embedding, ((0, TABLE_PAD - embedding.shape[0]), (0, 0)))
    table = pl.pallas_call(
        _mlp_body,
        out_shape=jax.ShapeDtypeStruct((TABLE_PAD, HIDDEN), jnp.float32),
    )(emb, W1, b1.reshape(1, HIDDEN), W2, b2.reshape(1, HIDDEN))

    idx = diffusion_step.astype(jnp.int32).reshape(NW, N_CHUNKS, CHUNK)
    gather = pl.kernel(
        _gather_body,
        out_type=jax.ShapeDtypeStruct((BATCH, HIDDEN), jnp.float32),
        mesh=plsc.VectorSubcoreMesh(core_axis_name="c", subcore_axis_name="s"),
        scratch_types=[
            pltpu.VMEM((N_CHUNKS, CHUNK), jnp.int32),
            pltpu.VMEM((CHUNK, HIDDEN), jnp.float32),
            pltpu.SemaphoreType.DMA,
        ],
    )
    return gather(table, idx)


# speedup vs baseline: 1.7017x; 1.0347x over previous
import jax
import jax.numpy as jnp
from jax import lax
from jax.experimental import pallas as pl
from jax.experimental.pallas import tpu as pltpu
from jax.experimental.pallas import tpu_sc as plsc

IN_DIM = 256
HIDDEN = 1024
TABLE_PAD = 1024
BATCH = 16384

NC, NS = 2, 16
NW = NC * NS
B_PER_W = BATCH // NW
CHUNK = 32
N_CHUNKS = B_PER_W // CHUNK
NBUF = 3


def _mlp_body(emb_ref, w1_ref, b1_ref, w2_ref, b2_ref, out_ref):
    h = jnp.dot(emb_ref[...], w1_ref[...], preferred_element_type=jnp.float32)
    h = h + b1_ref[...]
    h = h * jax.nn.sigmoid(h)
    o = jnp.dot(h, w2_ref[...], preferred_element_type=jnp.float32)
    o = o + b2_ref[...]
    out_ref[...] = o * jax.nn.sigmoid(o)


def _gather_body(table_hbm, idx_hbm, out_hbm, idx_v, rows_v,
                 g0, g1, g2, w0, w1, w2):
    gs, ws = [g0, g1, g2], [w0, w1, w2]
    wid = lax.axis_index("s") * NC + lax.axis_index("c")
    base = wid * B_PER_W
    pltpu.sync_copy(idx_hbm.at[wid], idx_v)
    g = [pltpu.async_copy(table_hbm.at[idx_v.at[b]], rows_v.at[b], gs[b])
         for b in range(NBUF)]
    w = [None] * NBUF
    for j in range(N_CHUNKS):
        b = j % NBUF
        g[b].wait()
        w[b] = pltpu.async_copy(rows_v.at[b],
                                out_hbm.at[pl.ds(base + j * CHUNK, CHUNK)],
                                ws[b])
        k = j + NBUF
        if k < N_CHUNKS:
            w[b].wait()
            g[b] = pltpu.async_copy(table_hbm.at[idx_v.at[k]], rows_v.at[b],
                                    gs[b])
    for j in range(max(0, N_CHUNKS - NBUF), N_CHUNKS):
        w[j % NBUF].wait()


def kernel(diffusion_step, embedding, W1, b1, W2, b2):
    emb = jnp.pad(embedding, ((0, TABLE_PAD - embedding.shape[0]), (0, 0)))
    table = pl.pallas_call(
        _mlp_body,
        out_shape=jax.ShapeDtypeStruct((TABLE_PAD, HIDDEN), jnp.float32),
    )(emb, W1, b1.reshape(1, HIDDEN), W2, b2.reshape(1, HIDDEN))

    idx = diffusion_step.astype(jnp.int32).reshape(NW, N_CHUNKS, CHUNK)
    gather = pl.kernel(
        _gather_body,
        out_type=jax.ShapeDtypeStruct((BATCH, HIDDEN), jnp.float32),
        mesh=plsc.VectorSubcoreMesh(core_axis_name="c", subcore_axis_name="s"),
        scratch_types=[
            pltpu.VMEM((N_CHUNKS, CHUNK), jnp.int32),
            pltpu.VMEM((NBUF, CHUNK, HIDDEN), jnp.float32),
            pltpu.SemaphoreType.DMA,
            pltpu.SemaphoreType.DMA,
            pltpu.SemaphoreType.DMA,
            pltpu.SemaphoreType.DMA,
            pltpu.SemaphoreType.DMA,
            pltpu.SemaphoreType.DMA,
        ],
    )
    return gather(table, idx)
